# baseline (speedup 1.0000x reference)
import functools

import jax
import jax.numpy as jnp
import numpy as np
from jax import lax
from jax.experimental import pallas as pl
from jax.experimental.pallas import tpu as pltpu
from jax.experimental.pallas import tpu_sc as plsc

N = 50000
E = 800000
F = 80
G = 128
NLAYERS = 4

_DEG = np.array([0, 50, 150, 300, 600, 1000, 1500, 2200, 3000, 3800, 4400, 4800, 5000, 4800, 4400, 3800, 3000, 2200, 1500, 1000, 600, 300, 150, 80, 40, 20, 10, 5, 2, 1, 0, 0, 0], dtype=np.float64)
_AVG = float((np.log(np.arange(_DEG.size) + 1.0) * _DEG).sum() / _DEG.sum())

NW = 32
WIN = 64
K = 128
NBUF = 8
FP = 96
FW = 48
NP = 50048
EPAD = E + NBUF * K + 8

CH = 3128
TCG = NP // CH
PCH = 2000


def _seg_stats(b_pk, spk, ptr, bounds):
    mesh = plsc.VectorSubcoreMesh(core_axis_name="c", subcore_axis_name="s",
                                  num_cores=2, num_subcores=16)
    outs = tuple(jax.ShapeDtypeStruct((NP, FP), jnp.float32) for _ in range(4))

    @functools.partial(
        pl.kernel, out_type=outs, mesh=mesh,
        compiler_params=pltpu.CompilerParams(use_tc_tiling_on_sc=False,
                                             needs_layout_passes=False),
        scratch_types=[
            pltpu.VMEM((NBUF, K), jnp.int32),
            pltpu.VMEM((NBUF, K), jnp.int32),
            pltpu.VMEM((NBUF * K, FW), jnp.int32),
            pltpu.VMEM((WIN, FP), jnp.float32),
            pltpu.VMEM((WIN, FP), jnp.float32),
            pltpu.VMEM((WIN, FP), jnp.float32),
            pltpu.VMEM((WIN, FP), jnp.float32),
            pltpu.VMEM((WIN + 16,), jnp.int32),
            pltpu.VMEM((48,), jnp.int32),
            pltpu.SemaphoreType.DMA((NBUF,)),
        ])
    def agg(b_hbm, spk_hbm, ptr_hbm, bnd_hbm, ssum_hbm, ssq_hbm, smn_hbm, smx_hbm,
            pk_v, idx_v, rows_v, ssum_v, ssq_v, smn_v, smx_v, ptr_s, bnd_s, sems):
        wid = lax.axis_index("s") * 2 + lax.axis_index("c")
        pltpu.sync_copy(bnd_hbm, bnd_s)
        bv = bnd_s[pl.ds(wid, 16)]
        n0 = bv[0]
        n1 = bv[1]

        @pl.loop(n0, n1, step=WIN)
        def _window(w0):
            w0 = pl.multiple_of(w0, WIN)
            pltpu.sync_copy(ptr_hbm.at[pl.ds(w0, WIN + 16)], ptr_s)
            e_begin = ptr_s[pl.ds(0, 16)][0]
            e_end = ptr_s[pl.ds(WIN, 16)][0]
            ea = (e_begin // 8) * 8
            nc = (e_end - ea + (K - 1)) // K

            zv = jnp.zeros((16,), jnp.float32)

            @pl.loop(0, WIN)
            def _zero(n):
                for j in range(FP // 16):
                    sl = pl.ds(16 * j, 16)
                    ssum_v[n, sl] = zv
                    ssq_v[n, sl] = zv
                    smn_v[n, sl] = zv
                    smx_v[n, sl] = zv

            def issue(c, b):
                e0 = pl.multiple_of(ea + c * K, 8)
                pltpu.sync_copy(spk_hbm.at[pl.ds(e0, K)], pk_v.at[b])
                for t in range(K // 16):
                    sl = pl.ds(16 * t, 16)
                    idx_v[b, sl] = pk_v[b, sl] & 0xFFFF
                pltpu.async_copy(b_hbm.at[idx_v.at[b]],
                                 rows_v.at[pl.ds(b * K, K)], sems.at[b])

            def wait(b):
                pltpu.make_async_copy(b_hbm.at[pl.ds(0, K)],
                                      rows_v.at[pl.ds(b * K, K)],
                                      sems.at[b]).wait()

            def process(c, b):
                chunk_lo = ea + c * K
                chunk_hi = chunk_lo + K

                @pl.loop(0, WIN)
                def _node(ln):
                    pv = ptr_s[pl.ds(ln, 16)]
                    e_lo = jnp.maximum(pv[0], chunk_lo)
                    e_hi = jnp.minimum(pv[1], chunk_hi)

                    @pl.when(e_lo < e_hi)
                    def _():
                        st = []
                        for j in range(6):
                            sl = pl.ds(16 * j, 16)
                            st += [ssum_v[ln, sl], ssq_v[ln, sl],
                                   smn_v[ln, sl], smx_v[ln, sl]]
                        fresh = pv[0] >= chunk_lo
                        big = jnp.full((16,), 3.0e38, jnp.float32)
                        for j in range(6):
                            st[4 * j + 2] = jnp.where(fresh, big, st[4 * j + 2])
                            st[4 * j + 3] = jnp.where(fresh, -big, st[4 * j + 3])

                        @pl.loop(e_lo, e_hi, init_carry=tuple(st))
                        def res(e, acc):
                            acc = list(acc)
                            p = e - chunk_lo + b * K
                            for t in range(3):
                                v = rows_v[p, pl.ds(16 * t, 16)]
                                for half in range(2):
                                    j = 2 * t + half
                                    if half == 0:
                                        r = plsc.bitcast(v << 16, jnp.float32)
                                    else:
                                        r = plsc.bitcast(
                                            v & jnp.int32(-65536), jnp.float32)
                                    acc[4 * j] = acc[4 * j] + r
                                    acc[4 * j + 1] = acc[4 * j + 1] + r * r
                                    acc[4 * j + 2] = jnp.minimum(acc[4 * j + 2], r)
                                    acc[4 * j + 3] = jnp.maximum(acc[4 * j + 3], r)
                            return tuple(acc)

                        for j in range(6):
                            sl = pl.ds(16 * j, 16)
                            ssum_v[ln, sl] = res[4 * j]
                            ssq_v[ln, sl] = res[4 * j + 1]
                            smn_v[ln, sl] = res[4 * j + 2]
                            smx_v[ln, sl] = res[4 * j + 3]

            for b in range(NBUF):
                @pl.when(b < nc)
                def _(b=b):
                    issue(jnp.int32(b), b)

            @pl.loop(0, nc, step=NBUF)
            def _grp(g):
                for b in range(NBUF):
                    c = g + b

                    @pl.when(c < nc)
                    def _(c=c, b=b):
                        wait(b)
                        process(c, b)

                        @pl.when(c + NBUF < nc)
                        def _(c=c, b=b):
                            issue(c + NBUF, b)

            pltpu.sync_copy(ssum_v, ssum_hbm.at[pl.ds(w0, WIN)])
            pltpu.sync_copy(ssq_v, ssq_hbm.at[pl.ds(w0, WIN)])
            pltpu.sync_copy(smn_v, smn_hbm.at[pl.ds(w0, WIN)])
            pltpu.sync_copy(smx_v, smx_hbm.at[pl.ds(w0, WIN)])

    return agg(b_pk, spk, ptr, bounds)


def _pack_b(hn, wlo_ref, whi_ref, blo_ref, bhi_ref):
    blo = hn @ wlo_ref[...] + blo_ref[...]
    bhi = hn @ whi_ref[...] + bhi_ref[...]
    lu = lax.bitcast_convert_type(blo, jnp.uint32)
    hu = lax.bitcast_convert_type(bhi, jnp.uint32)
    lu = (lu + jnp.uint32(0x8000)) >> 16
    hu = (hu + jnp.uint32(0x8000)) & jnp.uint32(0xFFFF0000)
    return lax.bitcast_convert_type(lu | hu, jnp.int32)


def _enc_pre(xp, Demb, base, Wa, Wlo, Whi, bqlo, bqhi):
    def body(x_ref, d_ref, base_ref, wa_ref, wlo_ref, whi_ref, blo_ref, bhi_ref,
             h_ref, a_ref, b_ref):
        xf = x_ref[...].astype(jnp.float32)
        h = xf @ d_ref[...] + base_ref[...]
        h_ref[...] = h
        a_ref[...] = h @ wa_ref[...]
        b_ref[...] = _pack_b(h, wlo_ref, whi_ref, blo_ref, bhi_ref)

    full = lambda s: pl.BlockSpec(s, lambda i: (0, 0))
    return pl.pallas_call(
        body,
        grid=(TCG,),
        in_specs=[
            pl.BlockSpec((CH, 9), lambda i: (i, 0)),
            full((9, F)), full((1, F)), full((F, FP)),
            full((F, FW)), full((F, FW)), full((1, FW)), full((1, FW)),
        ],
        out_specs=[pl.BlockSpec((CH, F), lambda i: (i, 0)),
                   pl.BlockSpec((CH, FP), lambda i: (i, 0)),
                   pl.BlockSpec((CH, FW), lambda i: (i, 0))],
        out_shape=[jax.ShapeDtypeStruct((NP, F), jnp.float32),
                   jax.ShapeDtypeStruct((NP, FP), jnp.float32),
                   jax.ShapeDtypeStruct((NP, FW), jnp.int32)],
    )(xp, Demb, base, Wa, Wlo, Whi, bqlo, bqhi)


def _post(h, a, sb, sb2, smn, smx, ptrA, ptrB, P0, Q1, Q2, Q3, W2, b2,
          Wa, Wlo, Whi, bqlo, bqhi, *, make_pre):
    def body(h_ref, a_ref, sb_ref, sb2_ref, smn_ref, smx_ref, pa_ref, pb_ref,
             p0_ref, q1_ref, q2_ref, q3_ref, w2_ref, b2_ref,
             wa_ref, wlo_ref, whi_ref, blo_ref, bhi_ref,
             ho_ref, ao_ref, bo_ref):
        cnt = (pb_ref[...] - pa_ref[...]).astype(jnp.float32)
        deg = jnp.maximum(cnt, 1.0)
        invdeg = 1.0 / deg
        cda = cnt * invdeg
        lg = jnp.log(deg + 1.0) * (1.0 / _AVG)
        att = 1.0 / lg
        has = (cnt > 0.0).astype(jnp.float32)
        h = h_ref[...]
        av = a_ref[...]
        sbv = sb_ref[...] * invdeg
        mean = cda * av + sbv
        ex2 = cda * (av * av) + 2.0 * av * sbv + sb2_ref[...] * invdeg
        var = jnp.maximum(ex2 - mean * mean, 0.0)
        std = jnp.sqrt(var + 1e-5)
        mn = has * (av + smn_ref[...])
        mx = has * (av + smx_ref[...])
        agg = jnp.concatenate([mean, mn, mx, std], axis=-1)
        z = (h @ p0_ref[...] + agg @ q1_ref[...]
             + lg * (agg @ q2_ref[...]) + att * (agg @ q3_ref[...]))
        out = jnp.maximum(z @ w2_ref[...] + b2_ref[...], 0.0)
        hn = h + out
        ho_ref[...] = hn
        if make_pre:
            ao_ref[...] = hn @ wa_ref[...]
            bo_ref[...] = _pack_b(hn, wlo_ref, whi_ref, blo_ref, bhi_ref)

    full = lambda s: pl.BlockSpec(s, lambda i: (0, 0))
    hblk = pl.BlockSpec((CH, F), lambda i: (i, 0))
    sblk = pl.BlockSpec((CH, FP), lambda i: (i, 0))
    iblk = pl.BlockSpec((CH, 1), lambda i: (i, 0))
    n_out = 3 if make_pre else 1
    return pl.pallas_call(
        body,
        grid=(TCG,),
        in_specs=[hblk, sblk, sblk, sblk, sblk, sblk, iblk, iblk,
                  full((F, F)), full((4 * FP, F)), full((4 * FP, F)),
                  full((4 * FP, F)), full((F, F)), full((1, F)),
                  full((F, FP)), full((F, FW)), full((F, FW)),
                  full((1, FW)), full((1, FW))],
        out_specs=[hblk, sblk, pl.BlockSpec((CH, FW), lambda i: (i, 0))],
        out_shape=[jax.ShapeDtypeStruct((NP, F), jnp.float32),
                   jax.ShapeDtypeStruct((NP, FP), jnp.float32),
                   jax.ShapeDtypeStruct((NP, FW), jnp.int32)],
    )(h, a, sb, sb2, smn, smx, ptrA, ptrB, P0, Q1, Q2, Q3, W2, b2,
      Wa, Wlo, Whi, bqlo, bqhi)[:n_out]


def _pool(h, batch2, Wmlp, bmlp):
    def body(h_ref, b_ref, wm_ref, bm_ref, o_ref, acc_ref, cnt_ref):
        i = pl.program_id(0)

        @pl.when(i == 0)
        def _():
            acc_ref[...] = jnp.zeros_like(acc_ref)
            cnt_ref[...] = jnp.zeros_like(cnt_ref)

        onehot = (b_ref[...] == lax.broadcasted_iota(jnp.int32, (1, G), 1)
                  ).astype(jnp.float32)
        acc_ref[...] += lax.dot_general(onehot, h_ref[...],
                                        (((0,), (0,)), ((), ())),
                                        preferred_element_type=jnp.float32)
        cnt_ref[...] += lax.dot_general(onehot, jnp.ones((PCH, 1), jnp.float32),
                                        (((0,), (0,)), ((), ())),
                                        preferred_element_type=jnp.float32)

        @pl.when(i == (N // PCH) - 1)
        def _():
            pooled = acc_ref[...] / jnp.maximum(cnt_ref[...], 1.0)
            o_ref[...] = pooled @ wm_ref[...] + bm_ref[...]

    return pl.pallas_call(
        body,
        grid=(N // PCH,),
        in_specs=[pl.BlockSpec((PCH, F), lambda i: (i, 0)),
                  pl.BlockSpec((PCH, 1), lambda i: (i, 0)),
                  pl.BlockSpec((F, 1), lambda i: (0, 0)),
                  pl.BlockSpec((1, 1), lambda i: (0, 0))],
        out_specs=pl.BlockSpec((G, 1), lambda i: (0, 0)),
        out_shape=jax.ShapeDtypeStruct((G, 1), jnp.float32),
        scratch_shapes=[pltpu.VMEM((G, F), jnp.float32),
                        pltpu.VMEM((G, 1), jnp.float32)],
    )(h, batch2, Wmlp, bmlp)


def kernel(x, edge_index, batch, emb, Wpre, bpre, Wpost, bpost, Wlin, blin,
           gamma, beta, Wmlp, bmlp):
    src = edge_index[0].astype(jnp.uint32)
    dst = edge_index[1].astype(jnp.uint32)

    spk_u = jnp.sort((dst << 16) | src)
    spk = jnp.zeros((EPAD,), jnp.int32).at[:E].set(
        lax.bitcast_convert_type(spk_u, jnp.int32))
    counts = jnp.zeros((NP,), jnp.int32).at[edge_index[1].astype(jnp.int32)].add(1)
    ptr = jnp.concatenate([jnp.zeros((1,), jnp.int32),
                           jnp.cumsum(counts, dtype=jnp.int32),
                           jnp.full((15,), E, jnp.int32)])
    targets = (jnp.arange(33, dtype=jnp.int32) * (E // NW)).astype(jnp.int32)
    bnd = jnp.searchsorted(ptr[:NP + 1], targets, side="left").astype(jnp.int32)
    bnd = (bnd // WIN) * WIN
    bnd = bnd.at[0].set(0).at[32].set(NP)
    bnd = lax.cummax(bnd)
    bounds = jnp.concatenate([bnd, jnp.zeros((15,), jnp.int32)])
    ptrA = ptr[:NP, None]
    ptrB = ptr[1:NP + 1, None]

    base = emb[:, 0, :].sum(0)[None, :]
    Demb = emb[:, 1, :] - emb[:, 0, :]
    sbn = gamma / np.sqrt(1.0 + 1e-5)
    Wa = jnp.pad(Wpre[:, :F, :], ((0, 0), (0, 0), (0, FP - F)))
    Wb = Wpre[:, F:, :]
    Wlo = jnp.concatenate([Wb[:, :, 0:16], Wb[:, :, 32:48], Wb[:, :, 64:80]], axis=2)
    Whi = jnp.concatenate([Wb[:, :, 16:32], Wb[:, :, 48:64],
                           jnp.zeros((NLAYERS, F, 16), jnp.float32)], axis=2)
    bqlo = jnp.concatenate([bpre[:, 0:16], bpre[:, 32:48], bpre[:, 64:80]], axis=1)
    bqhi = jnp.concatenate([bpre[:, 16:32], bpre[:, 48:64],
                            jnp.zeros((NLAYERS, 16), jnp.float32)], axis=1)
    P0 = Wpost[:, :F, :]
    Qn = Wpost[:, F:, :].reshape(NLAYERS, 12, F, F)
    Qp = jnp.pad(Qn, ((0, 0), (0, 0), (0, FP - F), (0, 0)))
    Q1 = Qp[:, 0:4].reshape(NLAYERS, 4 * FP, F)
    Q2 = Qp[:, 4:8].reshape(NLAYERS, 4 * FP, F)
    Q3 = Qp[:, 8:12].reshape(NLAYERS, 4 * FP, F)
    W2 = Wlin * sbn[:, None, :]
    b2 = (jnp.einsum("lf,lfg->lg", bpost, Wlin) + blin) * sbn + beta

    xp = jnp.zeros((NP, 9), jnp.int32).at[:N].set(x.astype(jnp.int32))

    h, a, b = _enc_pre(xp, Demb, base, Wa[0], Wlo[0], Whi[0],
                       bqlo[0][None, :], bqhi[0][None, :])
    for l in range(NLAYERS):
        sb, sb2, smn, smx = _seg_stats(b, spk, ptr, bounds)
        mp = l < NLAYERS - 1
        ln = l + 1 if mp else l
        res = _post(h, a, sb, sb2, smn, smx, ptrA, ptrB,
                    P0[l], Q1[l], Q2[l], Q3[l], W2[l], b2[l][None, :],
                    Wa[ln], Wlo[ln], Whi[ln],
                    bqlo[ln][None, :], bqhi[ln][None, :], make_pre=mp)
        if mp:
            h, a, b = res
        else:
            h = res[0]

    return _pool(h[:N], batch[:, None].astype(jnp.int32),
                 Wmlp, bmlp[None, :])
